# baseline (speedup 1.0000x reference)
import jax
import jax.numpy as jnp
from jax.experimental import pallas as pl
from jax.experimental.pallas import tpu as pltpu

N_EXPERTS = 64
K = 8
BT = 4096
DH = 384


def _router_block(xa_ref, xb_ref, wt_ref, b_ref, p_ref, m_ref):
    wt = wt_ref[...]
    logits = (
        jnp.dot(xa_ref[...], wt[:DH], preferred_element_type=jnp.float32)
        + jnp.dot(xb_ref[...], wt[DH:], preferred_element_type=jnp.float32)
        + b_ref[...]
    )
    e = jnp.exp(logits)
    s = jnp.sum(e, axis=-1, keepdims=True)
    p = e / s
    w = logits
    for _ in range(K - 1):
        m = jnp.max(w, axis=-1, keepdims=True)
        w = jnp.where(w == m, -jnp.inf, w)
    t = jnp.max(w, axis=-1, keepdims=True)
    p_ref[...] = p
    m_ref[...] = logits >= t


@jax.jit
def kernel(x, W, b):
    n_tokens, d_model = x.shape
    wt = W.T
    b2 = b.reshape(1, N_EXPERTS)
    probs, mask = pl.pallas_call(
        _router_block,
        grid=(n_tokens // BT,),
        in_specs=[
            pl.BlockSpec((BT, DH), lambda i: (i, 0)),
            pl.BlockSpec((BT, DH), lambda i: (i, 1)),
            pl.BlockSpec((d_model, N_EXPERTS), lambda i: (0, 0)),
            pl.BlockSpec((1, N_EXPERTS), lambda i: (0, 0)),
        ],
        out_specs=[
            pl.BlockSpec((BT, N_EXPERTS), lambda i: (i, 0)),
            pl.BlockSpec((BT, N_EXPERTS), lambda i: (i, 0)),
        ],
        out_shape=[
            jax.ShapeDtypeStruct((n_tokens, N_EXPERTS), jnp.float32),
            jax.ShapeDtypeStruct((n_tokens, N_EXPERTS), jnp.bool_),
        ],
        compiler_params=pltpu.CompilerParams(
            dimension_semantics=("parallel",),
        ),
    )(x, x, wt, b2)
    return (probs, mask)
